# baseline (speedup 1.0000x reference)
import functools

import jax
import jax.numpy as jnp
from jax import lax
from jax.experimental import pallas as pl
from jax.experimental.pallas import tpu as pltpu

_K = 512
_D = 96
_ROWS = 8 * 96 * 110 * 110 // _D
_TC = 2200
_STEPS = _ROWS // _TC


def _vq_body(ft_ref, cb_ref, cbT_ref, c2_ref, a2_ref, qt_ref, loss_ref,
             perp_ref, sse_ref, counts_ref):
    i = pl.program_id(0)
    a = ft_ref[0, :, :]
    prod = jnp.dot(cb_ref[:, :], a, preferred_element_type=jnp.float32)
    d2 = jnp.sqrt(jnp.maximum(a2_ref[0, :, :] - 2.0 * prod + c2_ref[:, :],
                              0.0))
    m = jnp.min(d2, axis=0, keepdims=True)
    row = lax.broadcasted_iota(jnp.int32, d2.shape, 0)
    idx = jnp.min(jnp.where(d2 == m, row, _K), axis=0, keepdims=True)
    oh = (row == idx).astype(jnp.float32)
    qt_ref[0, :, :] = jnp.dot(cbT_ref[:, :], oh,
                              preferred_element_type=jnp.float32)
    sse_t = jnp.reshape(jnp.sum(m * m), (1, 1))
    cnt_t = jnp.sum(oh, axis=1, keepdims=True)

    @pl.when(i == 0)
    def _init():
        sse_ref[:, :] = jnp.zeros_like(sse_ref)
        counts_ref[:, :] = jnp.zeros_like(counts_ref)

    sse_ref[:, :] += sse_t
    counts_ref[:, :] += cnt_t

    @pl.when(i == _STEPS - 1)
    def _fin():
        mse = sse_ref[0, 0] / jnp.float32(_ROWS * _D)
        loss_ref[:, :] = jnp.reshape(1.25 * mse, (1, 1))
        p = counts_ref[:, :] / jnp.float32(_ROWS)
        ent = jnp.sum(p * jnp.log(p + 1e-10))
        perp_ref[:, :] = jnp.reshape(jnp.exp(-ent), (1, 1))


def _vq_t(flat_t, a2_t, codebook):
    cbT = codebook.T
    c2 = jnp.sum(codebook ** 2, axis=1)[:, None]
    qt, loss, perp = pl.pallas_call(
        _vq_body,
        grid=(_STEPS,),
        in_specs=[
            pl.BlockSpec((1, _D, _TC), lambda i: (i, 0, 0)),
            pl.BlockSpec((_K, _D), lambda i: (0, 0)),
            pl.BlockSpec((_D, _K), lambda i: (0, 0)),
            pl.BlockSpec((_K, 1), lambda i: (0, 0)),
            pl.BlockSpec((1, 1, _TC), lambda i: (i, 0, 0)),
        ],
        out_specs=[
            pl.BlockSpec((1, _D, _TC), lambda i: (i, 0, 0)),
            pl.BlockSpec((1, 1), lambda i: (0, 0)),
            pl.BlockSpec((1, 1), lambda i: (0, 0)),
        ],
        out_shape=[
            jax.ShapeDtypeStruct((_STEPS, _D, _TC), jnp.float32),
            jax.ShapeDtypeStruct((1, 1), jnp.float32),
            jax.ShapeDtypeStruct((1, 1), jnp.float32),
        ],
        scratch_shapes=[
            pltpu.VMEM((1, 1), jnp.float32),
            pltpu.VMEM((_K, 1), jnp.float32),
        ],
        compiler_params=pltpu.CompilerParams(
            dimension_semantics=("arbitrary",),
        ),
    )(flat_t, codebook, cbT, c2, a2_t)
    return qt, loss[0, 0], perp[0, 0]


def _conv(x, w, b, stride):
    y = lax.conv_general_dilated(x, w, (stride, stride), 'VALID',
                                 dimension_numbers=('NCHW', 'OIHW', 'NCHW'))
    return y + b[None, :, None, None]


def _deconv(x, w, b, stride):
    y = lax.conv_transpose(x, w, (stride, stride), 'VALID',
                           dimension_numbers=('NCHW', 'OIHW', 'NCHW'),
                           transpose_kernel=True)
    return y + b[None, :, None, None]


def kernel(x, conv1_w, conv1_b, conv2_w, conv2_b, codebook,
           deconv1_w, deconv1_b, deconv2_w, deconv2_b):
    z = jax.nn.relu(_conv(x, conv1_w, conv1_b, 2))
    z = jax.nn.relu(_conv(z, conv2_w, conv2_b, 1))
    flat = z.reshape(-1, _D)
    a2 = jnp.sum(flat ** 2, axis=1, keepdims=True)
    one = 1.0 + 0.0 * jnp.sum(conv1_b)
    flat3 = flat.reshape(_STEPS, _TC, _D).swapaxes(1, 2) * one
    a23 = jnp.reshape(a2, (_STEPS, 1, _TC))
    qt, loss, perp = _vq_t(flat3, a23, codebook)
    quantized = (qt.swapaxes(1, 2) * one).reshape(z.shape)
    h = jax.nn.relu(_deconv(quantized, deconv1_w, deconv1_b, 1))
    x_recon = _deconv(h, deconv2_w, deconv2_b, 2)
    return (x_recon, loss, perp)


# speedup vs baseline: 1.0096x; 1.0096x over previous
import functools

import jax
import jax.numpy as jnp
from jax import lax
from jax.experimental import pallas as pl
from jax.experimental.pallas import tpu as pltpu

_K = 512
_D = 96
_ROWS = 8 * 96 * 110 * 110 // _D
_TC = 2200
_STEPS = _ROWS // _TC


def _vq_body(ft_ref, cb_ref, cbT_ref, c2_ref, a2_ref, qt_ref, loss_ref,
             perp_ref, sse_ref, counts_ref):
    i = pl.program_id(0)
    a = ft_ref[0, :, :]
    prod = jnp.dot(cb_ref[:, :].astype(jnp.bfloat16), a,
                   preferred_element_type=jnp.float32)
    d2 = jnp.sqrt(jnp.maximum(a2_ref[0, :, :] - 2.0 * prod + c2_ref[:, :],
                              0.0))
    m = jnp.min(d2, axis=0, keepdims=True)
    row = lax.broadcasted_iota(jnp.int32, d2.shape, 0)
    idx = jnp.min(jnp.where(d2 == m, row, _K), axis=0, keepdims=True)
    oh = (row == idx).astype(jnp.float32)
    q32 = jnp.dot(cbT_ref[:, :].astype(jnp.bfloat16), oh.astype(jnp.bfloat16),
                  preferred_element_type=jnp.float32)
    qt_ref[0, :, :] = q32.astype(jnp.bfloat16)
    sse_t = jnp.reshape(jnp.sum(m * m), (1, 1))
    cnt_t = jnp.sum(oh, axis=1, keepdims=True)

    @pl.when(i == 0)
    def _init():
        sse_ref[:, :] = jnp.zeros_like(sse_ref)
        counts_ref[:, :] = jnp.zeros_like(counts_ref)

    sse_ref[:, :] += sse_t
    counts_ref[:, :] += cnt_t

    @pl.when(i == _STEPS - 1)
    def _fin():
        mse = sse_ref[0, 0] / jnp.float32(_ROWS * _D)
        loss_ref[:, :] = jnp.reshape(1.25 * mse, (1, 1))
        p = counts_ref[:, :] / jnp.float32(_ROWS)
        ent = jnp.sum(p * jnp.log(p + 1e-10))
        perp_ref[:, :] = jnp.reshape(jnp.exp(-ent), (1, 1))


def _vq_t(flat_t, a2_t, codebook):
    cbT = codebook.T
    c2 = jnp.sum(codebook ** 2, axis=1)[:, None]
    qt, loss, perp = pl.pallas_call(
        _vq_body,
        grid=(_STEPS,),
        in_specs=[
            pl.BlockSpec((1, _D, _TC), lambda i: (i, 0, 0)),
            pl.BlockSpec((_K, _D), lambda i: (0, 0)),
            pl.BlockSpec((_D, _K), lambda i: (0, 0)),
            pl.BlockSpec((_K, 1), lambda i: (0, 0)),
            pl.BlockSpec((1, 1, _TC), lambda i: (i, 0, 0)),
        ],
        out_specs=[
            pl.BlockSpec((1, _D, _TC), lambda i: (i, 0, 0)),
            pl.BlockSpec((1, 1), lambda i: (0, 0)),
            pl.BlockSpec((1, 1), lambda i: (0, 0)),
        ],
        out_shape=[
            jax.ShapeDtypeStruct((_STEPS, _D, _TC), jnp.bfloat16),
            jax.ShapeDtypeStruct((1, 1), jnp.float32),
            jax.ShapeDtypeStruct((1, 1), jnp.float32),
        ],
        scratch_shapes=[
            pltpu.VMEM((1, 1), jnp.float32),
            pltpu.VMEM((_K, 1), jnp.float32),
        ],
        compiler_params=pltpu.CompilerParams(
            dimension_semantics=("arbitrary",),
        ),
    )(flat_t, codebook, cbT, c2, a2_t)
    return qt, loss[0, 0], perp[0, 0]


def _conv(x, w, b, stride):
    y = lax.conv_general_dilated(x, w, (stride, stride), 'VALID',
                                 dimension_numbers=('NCHW', 'OIHW', 'NCHW'))
    return y + b[None, :, None, None]


def _deconv(x, w, b, stride):
    y = lax.conv_transpose(x, w, (stride, stride), 'VALID',
                           dimension_numbers=('NCHW', 'OIHW', 'NCHW'),
                           transpose_kernel=True)
    return y + b[None, :, None, None]


def kernel(x, conv1_w, conv1_b, conv2_w, conv2_b, codebook,
           deconv1_w, deconv1_b, deconv2_w, deconv2_b):
    z = jax.nn.relu(_conv(x, conv1_w, conv1_b, 2))
    z = jax.nn.relu(_conv(z, conv2_w, conv2_b, 1))
    flat = z.reshape(-1, _D)
    a2 = jnp.sum(flat ** 2, axis=1, keepdims=True)
    flat3 = flat.reshape(_STEPS, _TC, _D).astype(jnp.bfloat16).swapaxes(1, 2)
    a23 = jnp.reshape(a2, (_STEPS, 1, _TC))
    qt, loss, perp = _vq_t(flat3, a23, codebook)
    quantized = qt.swapaxes(1, 2).reshape(z.shape).astype(jnp.float32)
    h = jax.nn.relu(_deconv(quantized, deconv1_w, deconv1_b, 1))
    x_recon = _deconv(h, deconv2_w, deconv2_b, 2)
    return (x_recon, loss, perp)
